# Initial kernel scaffold; baseline (speedup 1.0000x reference)
import jax
import jax.numpy as jnp
from jax.experimental import pallas as pl


def kernel(x, edge_index, prompt_sim_pt, prompt_degree_pt, prompt_other_pt, readout_token, in_proj_w, in_proj_b, out_proj_w, out_proj_b):
    raise NotImplementedError("write your pallas kernel here")


# speedup vs baseline: 1.5022x; 1.5022x over previous
import functools

import jax
import jax.numpy as jnp
from jax import lax
from jax.experimental import pallas as pl
from jax.experimental.pallas import tpu as pltpu
from jax.experimental.pallas import tpu_sc as plsc

N = 10000
E = 320000
D = 128
SIM_T = 0.5
DEG_T = 2.0

NC = 2
NS = 16
NW = NC * NS
EPW = E // NW
CHUNK = 80
NCHUNK = EPW // CHUNK
GROUPS = CHUNK // 16

ROW_BLK = 1000


def _norm_body(x_ref, o_ref):
    xb = x_ref[...]
    s = jnp.sum(xb * xb, axis=1, keepdims=True)
    o_ref[...] = xb / jnp.sqrt(s)


def _normalize(x):
    return pl.pallas_call(
        _norm_body,
        grid=(N // ROW_BLK,),
        in_specs=[pl.BlockSpec((ROW_BLK, D), lambda i: (i, 0))],
        out_specs=pl.BlockSpec((ROW_BLK, D), lambda i: (i, 0)),
        out_shape=jax.ShapeDtypeStruct((N, D), jnp.float32),
    )(x)


def _sc_edge_body(xn_hbm, row_hbm, col_hbm, c_out, deg_out,
                  rowi_v, coli_v, u_v, v_v, c_p, deg_p, sem_u, sem_v):
    cid = lax.axis_index("c")
    sid = lax.axis_index("s")
    wid = sid * NC + cid
    base = wid * EPW

    zero16 = jnp.zeros((16,), jnp.float32)
    ones16 = jnp.ones((16,), jnp.float32)
    lanes = lax.iota(jnp.int32, 16)

    def _zero(i, carry):
        c_p[pl.ds(i * 16, 16)] = zero16
        deg_p[pl.ds(i * 16, 16)] = zero16
        return carry
    lax.fori_loop(0, N // 16, _zero, 0)

    def _chunk(g, carry):
        cbase = base + g * CHUNK
        pltpu.sync_copy(row_hbm.at[pl.ds(cbase, CHUNK)], rowi_v)
        pltpu.sync_copy(col_hbm.at[pl.ds(cbase, CHUNK)], coli_v)
        du = pltpu.async_copy(xn_hbm.at[rowi_v], u_v, sem_u)
        dv = pltpu.async_copy(xn_hbm.at[coli_v], v_v, sem_v)
        du.wait()
        dv.wait()
        for grp in range(GROUPS):
            elanes = lanes + (grp * 16)

            def _dstep(j, acc):
                for u8 in range(8):
                    dvec = jnp.broadcast_to(j * 8 + u8, (16,))
                    uu = plsc.load_gather(u_v, [elanes, dvec])
                    vv = plsc.load_gather(v_v, [elanes, dvec])
                    acc = acc + uu * vv
                return acc

            acc = lax.fori_loop(0, D // 8, _dstep, zero16)
            cols = coli_v[pl.ds(grp * 16, 16)]
            plsc.addupdate_scatter(c_p, [cols], acc)
            plsc.addupdate_scatter(deg_p, [cols], ones16)
        return carry

    lax.fori_loop(0, NCHUNK, _chunk, 0)

    pltpu.sync_copy(c_p, c_out.at[wid])
    pltpu.sync_copy(deg_p, deg_out.at[wid])


_edge_call = pl.kernel(
    _sc_edge_body,
    out_type=[
        jax.ShapeDtypeStruct((NW, N), jnp.float32),
        jax.ShapeDtypeStruct((NW, N), jnp.float32),
    ],
    mesh=plsc.VectorSubcoreMesh(core_axis_name="c", subcore_axis_name="s"),
    compiler_params=pltpu.CompilerParams(needs_layout_passes=False),
    scratch_types=[
        pltpu.VMEM((CHUNK,), jnp.int32),
        pltpu.VMEM((CHUNK,), jnp.int32),
        pltpu.VMEM((CHUNK, D), jnp.float32),
        pltpu.VMEM((CHUNK, D), jnp.float32),
        pltpu.VMEM((N,), jnp.float32),
        pltpu.VMEM((N,), jnp.float32),
        pltpu.SemaphoreType.DMA,
        pltpu.SemaphoreType.DMA,
    ],
)


def _final_body(x_ref, c_ref, dg_ref, psim_ref, pdeg_ref, poth_ref, ro_ref,
                wi_ref, bi_ref, wo_ref, bo_ref, o_ref):
    c = jnp.sum(c_ref[...], axis=1, keepdims=True)
    deg = jnp.sum(dg_ref[...], axis=1, keepdims=True)
    csim = c / deg
    sim_m = csim <= SIM_T
    deg_m = deg <= DEG_T
    not_sim = jnp.logical_not(sim_m)
    not_deg = jnp.logical_not(deg_m)
    f32 = lambda m: m.astype(jnp.float32)
    S = jnp.concatenate([
        f32(jnp.logical_and(sim_m, deg_m)),
        f32(jnp.logical_and(sim_m, not_deg)),
        f32(jnp.logical_and(not_sim, deg_m)),
        f32(jnp.logical_and(not_sim, not_deg)),
    ], axis=1)

    ro = ro_ref[...]
    ps = psim_ref[...]
    pd = pdeg_ref[...]
    po = poth_ref[...]
    z = jnp.zeros_like(ro)
    toks = jnp.concatenate([
        ro, ps, pd, z,
        ro, ps, z, z,
        ro, z, pd, z,
        ro, z, z, po,
    ], axis=0)
    bi = bi_ref[...]
    qkv = lax.dot_general(toks, wi_ref[...], (((1,), (1,)), ((), ())),
                          preferred_element_type=jnp.float32) + bi
    kk = qkv[:, D:2 * D]
    vv = qkv[:, 2 * D:]
    qv = qkv[0:1, :D]
    svec = lax.dot_general(qv, kk, (((1,), (1,)), ((), ())),
                           preferred_element_type=jnp.float32) / (D ** 0.5)
    svec_b = jnp.broadcast_to(svec, (4, 16))
    col_grp = lax.broadcasted_iota(jnp.int32, (4, 16), 1) // 4
    row_id = lax.broadcasted_iota(jnp.int32, (4, 16), 0)
    sm = jnp.where(col_grp == row_id, svec_b, -1e30)
    attn = jax.nn.softmax(sm, axis=-1)
    ctx = lax.dot_general(attn, vv, (((1,), (0,)), ((), ())),
                          preferred_element_type=jnp.float32)
    tbl = lax.dot_general(ctx, wo_ref[...], (((1,), (1,)), ((), ())),
                          preferred_element_type=jnp.float32) + bo_ref[...]
    prompt = lax.dot_general(S, tbl, (((1,), (0,)), ((), ())),
                             preferred_element_type=jnp.float32)
    o_ref[...] = x_ref[...] + prompt


def _finalize(x, c_parts, deg_parts, psim, pdeg, poth, ro, wi, bi, wo, bo):
    full = lambda shape: pl.BlockSpec(shape, lambda i: tuple(0 for _ in shape))
    return pl.pallas_call(
        _final_body,
        grid=(N // ROW_BLK,),
        in_specs=[
            pl.BlockSpec((ROW_BLK, D), lambda i: (i, 0)),
            pl.BlockSpec((ROW_BLK, NW), lambda i: (i, 0)),
            pl.BlockSpec((ROW_BLK, NW), lambda i: (i, 0)),
            full((1, D)),
            full((1, D)),
            full((1, D)),
            full((1, D)),
            full((3 * D, D)),
            full((1, 3 * D)),
            full((D, D)),
            full((1, D)),
        ],
        out_specs=pl.BlockSpec((ROW_BLK, D), lambda i: (i, 0)),
        out_shape=jax.ShapeDtypeStruct((N, D), jnp.float32),
    )(x, c_parts, deg_parts, psim, pdeg, poth, ro, wi, bi, wo, bo)


@jax.jit
def kernel(x, edge_index, prompt_sim_pt, prompt_degree_pt, prompt_other_pt,
           readout_token, in_proj_w, in_proj_b, out_proj_w, out_proj_b):
    row = edge_index[0]
    col = edge_index[1]
    xn = _normalize(x)
    c_parts, deg_parts = _edge_call(xn, row, col)
    c_parts = c_parts.T
    deg_parts = deg_parts.T
    ro = readout_token.reshape(1, D)
    bi = in_proj_b.reshape(1, 3 * D)
    bo = out_proj_b.reshape(1, D)
    return _finalize(x, c_parts, deg_parts, prompt_sim_pt, prompt_degree_pt,
                     prompt_other_pt, ro, in_proj_w, bi, out_proj_w, bo)
